# baseline (speedup 1.0000x reference)
import functools

import jax
import jax.numpy as jnp
from jax import lax
from jax.experimental import pallas as pl
from jax.experimental.pallas import tpu as pltpu
from jax.experimental.pallas import tpu_sc as plsc

BATCH = 16384
D = 32
H1 = 64
H2 = 32
PK = 4
BP = BATCH // PK
TROWS = 1000000 // PK

NC = 2
NS = 16
NW = NC * NS
B_PER_W = BATCH // NW
R_PER_W = B_PER_W // PK
CHUNK = 128

_mesh = plsc.VectorSubcoreMesh(
    core_axis_name="c", subcore_axis_name="s", num_cores=NC, num_subcores=NS
)


@functools.partial(
    pl.kernel,
    out_type=(
        jax.ShapeDtypeStruct((BP * PK * D,), jnp.float32),
        jax.ShapeDtypeStruct((BP * PK * D,), jnp.float32),
    ),
    mesh=_mesh,
    compiler_params=pltpu.CompilerParams(use_tc_tiling_on_sc=True,
                                         needs_layout_passes=False),
    scratch_types=[
        pltpu.VMEM((B_PER_W,), jnp.int32),
        pltpu.VMEM((B_PER_W,), jnp.int32),
        pltpu.VMEM((B_PER_W,), jnp.int32),
        pltpu.VMEM((B_PER_W, PK * D), jnp.float32),
        pltpu.VMEM((B_PER_W * D,), jnp.float32),
        pltpu.SemaphoreType.DMA,
    ],
)
def _sc_gather2(uids_hbm, iids_hbm, utab_hbm, itab_hbm, uout_hbm, iout_hbm,
                ids_v, gidx_v, lane_v, gath_v, outst_v, sem):
    wid = lax.axis_index("s") * NC + lax.axis_index("c")
    base = wid * B_PER_W
    obase = wid * B_PER_W * D

    iota16 = lax.iota(jnp.int32, 16)

    def one_table(ids_hbm, tab_hbm, out_hbm):
        pltpu.sync_copy(ids_hbm.at[pl.ds(base, B_PER_W)], ids_v)

        def idx_body(g, _):
            goff = g * 16
            ids = ids_v[pl.ds(goff, 16)]
            gidx_v[pl.ds(goff, 16)] = lax.bitwise_or(
                lax.shift_left(lax.shift_right_logical(ids, 11), 9),
                lax.bitwise_and(ids, 511))
            lane_v[pl.ds(goff, 16)] = lax.bitwise_and(
                lax.shift_right_logical(ids, 9), 3)
            return 0

        lax.fori_loop(0, B_PER_W // 16, idx_body, 0)

        copies = [
            pltpu.async_copy(tab_hbm.at[gidx_v.at[pl.ds(k * CHUNK, CHUNK)]],
                             gath_v.at[pl.ds(k * CHUNK, CHUNK)], sem)
            for k in range(B_PER_W // CHUNK)
        ]
        for cp in copies:
            cp.wait()

        def ext_body(g, _):
            flat = g * 16 + iota16
            m = lax.shift_right_logical(flat, 5)
            e = lax.bitwise_and(flat, 31)
            ln = plsc.load_gather(lane_v, [m])
            val = plsc.load_gather(gath_v, [m, ln * D + e])
            outst_v[pl.ds(g * 16, 16)] = val
            return 0

        lax.fori_loop(0, (B_PER_W * D) // 16, ext_body, 0)
        pltpu.sync_copy(outst_v, out_hbm.at[pl.ds(obase, B_PER_W * D)])

    one_table(uids_hbm, utab_hbm, uout_hbm)
    one_table(iids_hbm, itab_hbm, iout_hbm)


def _mlp_body(u_ref, v_ref, w1a_ref, w1b_ref, b1_ref, w2_ref, b2_ref,
              w3_ref, b3_ref, o_ref):
    h = (jnp.dot(u_ref[...], w1a_ref[...], preferred_element_type=jnp.float32)
         + jnp.dot(v_ref[...], w1b_ref[...], preferred_element_type=jnp.float32)
         + b1_ref[...])
    h = jnp.maximum(h, 0.0)
    h = jnp.maximum(
        jnp.dot(h, w2_ref[...], preferred_element_type=jnp.float32) + b2_ref[...],
        0.0)
    o_ref[...] = (jnp.dot(h, w3_ref[...], preferred_element_type=jnp.float32)
                  + b3_ref[...])


_TW = 2048
_TGRID = 489
TROWS_PAD = _TGRID * (_TW // PK)


def _pack_body(x_ref, eye_ref, o_ref):
    x = x_ref[...]
    eye = eye_ref[...]
    q = _TW // PK
    o_ref[...] = jnp.concatenate(
        [lax.dot_general(x[:, k * q:(k + 1) * q], eye, (((0,), (0,)), ((), ())),
                         preferred_element_type=jnp.float32)
         for k in range(PK)], axis=1)


def _tc_pack(tab_t, eye32):
    return pl.pallas_call(
        _pack_body,
        grid=(_TGRID,),
        in_specs=[pl.BlockSpec((D, _TW), lambda i: (0, i)),
                  pl.BlockSpec((D, D), lambda i: (0, 0))],
        out_specs=pl.BlockSpec((_TW // PK, PK * D), lambda i: (i, 0)),
        out_shape=jax.ShapeDtypeStruct((TROWS_PAD, PK * D), jnp.float32),
    )(tab_t, eye32)


_BLK = 1024


def _tc_mlp(u, v, w1a, w1b, b1, w2, b2, w3, b3):
    grid = (BP // _BLK,)
    full = lambda i: (0, 0)
    return pl.pallas_call(
        _mlp_body,
        grid=grid,
        in_specs=[
            pl.BlockSpec((_BLK, PK * D), lambda i: (i, 0)),
            pl.BlockSpec((_BLK, PK * D), lambda i: (i, 0)),
            pl.BlockSpec((PK * D, PK * H1), full),
            pl.BlockSpec((PK * D, PK * H1), full),
            pl.BlockSpec((1, PK * H1), full),
            pl.BlockSpec((PK * H1, PK * H2), full),
            pl.BlockSpec((1, PK * H2), full),
            pl.BlockSpec((PK * H2, PK), full),
            pl.BlockSpec((1, PK), full),
        ],
        out_specs=pl.BlockSpec((_BLK, PK), lambda i: (i, 0)),
        out_shape=jax.ShapeDtypeStruct((BP, PK), jnp.float32),
    )(u, v, w1a, w1b, b1, w2, b2, w3, b3)


def kernel(user_ids, item_ids, user_table, item_table, W1, b1, W2, b2, W3, b3):
    eye32 = jnp.eye(D, dtype=jnp.float32)
    utab4 = _tc_pack(user_table.T, eye32)
    itab4 = _tc_pack(item_table.T, eye32)
    uflat, vflat = _sc_gather2(user_ids.astype(jnp.int32),
                               item_ids.astype(jnp.int32), utab4, itab4)
    u4 = uflat.reshape(BP, PK * D)
    v4 = vflat.reshape(BP, PK * D)
    eye = jnp.eye(PK, dtype=jnp.float32)
    o4 = _tc_mlp(u4, v4, jnp.kron(eye, W1[:D]), jnp.kron(eye, W1[D:]),
                 jnp.tile(b1, PK).reshape(1, PK * H1),
                 jnp.kron(eye, W2), jnp.tile(b2, PK).reshape(1, PK * H2),
                 jnp.kron(eye, W3), jnp.tile(b3, PK).reshape(1, PK))
    return o4.reshape(BATCH, 1)


# speedup vs baseline: 1.5805x; 1.5805x over previous
import functools

import jax
import jax.numpy as jnp
from jax import lax
from jax.experimental import pallas as pl
from jax.experimental.pallas import tpu as pltpu
from jax.experimental.pallas import tpu_sc as plsc

BATCH = 16384
D = 32
H1 = 64
H2 = 32
PK = 4
BP = BATCH // PK
TROWS = 1000000 // PK

NC = 2
NS = 16
NW = NC * NS
B_PER_W = BATCH // NW
R_PER_W = B_PER_W // PK
CHUNK = 128

_mesh = plsc.VectorSubcoreMesh(
    core_axis_name="c", subcore_axis_name="s", num_cores=NC, num_subcores=NS
)


@functools.partial(
    pl.kernel,
    out_type=(
        jax.ShapeDtypeStruct((BP * PK * D,), jnp.float32),
        jax.ShapeDtypeStruct((BP * PK * D,), jnp.float32),
    ),
    mesh=_mesh,
    compiler_params=pltpu.CompilerParams(use_tc_tiling_on_sc=True,
                                         needs_layout_passes=False),
    scratch_types=[
        pltpu.VMEM((B_PER_W,), jnp.int32),
        pltpu.VMEM((B_PER_W,), jnp.int32),
        pltpu.VMEM((B_PER_W,), jnp.int32),
        pltpu.VMEM((B_PER_W, PK * D), jnp.float32),
        pltpu.VMEM((B_PER_W * D,), jnp.float32),
        pltpu.SemaphoreType.DMA,
    ],
)
def _sc_gather2(uids_hbm, iids_hbm, utab_hbm, itab_hbm, uout_hbm, iout_hbm,
                ids_v, gidx_v, lane_v, gath_v, outst_v, sem):
    wid = lax.axis_index("s") * NC + lax.axis_index("c")
    base = wid * B_PER_W
    obase = wid * B_PER_W * D

    iota16 = lax.iota(jnp.int32, 16)

    def one_table(ids_hbm, tab_hbm, out_hbm):
        pltpu.sync_copy(ids_hbm.at[pl.ds(base, B_PER_W)], ids_v)

        def idx_body(g, _):
            goff = g * 16
            ids = ids_v[pl.ds(goff, 16)]
            gidx_v[pl.ds(goff, 16)] = lax.bitwise_or(
                lax.shift_left(lax.shift_right_logical(ids, 13), 11),
                lax.bitwise_and(ids, 2047))
            lane_v[pl.ds(goff, 16)] = lax.bitwise_and(
                lax.shift_right_logical(ids, 11), 3)
            return 0

        lax.fori_loop(0, B_PER_W // 16, idx_body, 0)

        copies = [
            pltpu.async_copy(tab_hbm.at[gidx_v.at[pl.ds(k * CHUNK, CHUNK)]],
                             gath_v.at[pl.ds(k * CHUNK, CHUNK)], sem)
            for k in range(B_PER_W // CHUNK)
        ]
        for cp in copies:
            cp.wait()

        def ext_body(g, _):
            flat = g * 16 + iota16
            m = lax.shift_right_logical(flat, 5)
            e = lax.bitwise_and(flat, 31)
            ln = plsc.load_gather(lane_v, [m])
            val = plsc.load_gather(gath_v, [m, ln * D + e])
            outst_v[pl.ds(g * 16, 16)] = val
            return 0

        lax.fori_loop(0, (B_PER_W * D) // 16, ext_body, 0)
        pltpu.sync_copy(outst_v, out_hbm.at[pl.ds(obase, B_PER_W * D)])

    one_table(uids_hbm, utab_hbm, uout_hbm)
    one_table(iids_hbm, itab_hbm, iout_hbm)


def _mlp_body(u_ref, v_ref, w1a_ref, w1b_ref, b1_ref, w2_ref, b2_ref,
              w3_ref, b3_ref, o_ref):
    h = (jnp.dot(u_ref[...], w1a_ref[...], preferred_element_type=jnp.float32)
         + jnp.dot(v_ref[...], w1b_ref[...], preferred_element_type=jnp.float32)
         + b1_ref[...])
    h = jnp.maximum(h, 0.0)
    h = jnp.maximum(
        jnp.dot(h, w2_ref[...], preferred_element_type=jnp.float32) + b2_ref[...],
        0.0)
    o_ref[...] = (jnp.dot(h, w3_ref[...], preferred_element_type=jnp.float32)
                  + b3_ref[...])


_TW = 8192
_TGRID = 123
TROWS_PAD = _TGRID * (_TW // PK)


def _pack_body(x_ref, eye_ref, o_ref):
    x = x_ref[...]
    eye = eye_ref[...]
    q = _TW // PK
    o_ref[...] = jnp.concatenate(
        [lax.dot_general(x[:, k * q:(k + 1) * q], eye, (((0,), (0,)), ((), ())),
                         preferred_element_type=jnp.float32)
         for k in range(PK)], axis=1)


def _tc_pack(tab_t, eye32):
    return pl.pallas_call(
        _pack_body,
        grid=(_TGRID,),
        in_specs=[pl.BlockSpec((D, _TW), lambda i: (0, i)),
                  pl.BlockSpec((D, D), lambda i: (0, 0))],
        out_specs=pl.BlockSpec((_TW // PK, PK * D), lambda i: (i, 0)),
        out_shape=jax.ShapeDtypeStruct((TROWS_PAD, PK * D), jnp.float32),
    )(tab_t, eye32)


_BLK = 1024


def _tc_mlp(u, v, w1a, w1b, b1, w2, b2, w3, b3):
    grid = (BP // _BLK,)
    full = lambda i: (0, 0)
    return pl.pallas_call(
        _mlp_body,
        grid=grid,
        in_specs=[
            pl.BlockSpec((_BLK, PK * D), lambda i: (i, 0)),
            pl.BlockSpec((_BLK, PK * D), lambda i: (i, 0)),
            pl.BlockSpec((PK * D, PK * H1), full),
            pl.BlockSpec((PK * D, PK * H1), full),
            pl.BlockSpec((1, PK * H1), full),
            pl.BlockSpec((PK * H1, PK * H2), full),
            pl.BlockSpec((1, PK * H2), full),
            pl.BlockSpec((PK * H2, PK), full),
            pl.BlockSpec((1, PK), full),
        ],
        out_specs=pl.BlockSpec((_BLK, PK), lambda i: (i, 0)),
        out_shape=jax.ShapeDtypeStruct((BP, PK), jnp.float32),
    )(u, v, w1a, w1b, b1, w2, b2, w3, b3)


def kernel(user_ids, item_ids, user_table, item_table, W1, b1, W2, b2, W3, b3):
    eye32 = jnp.eye(D, dtype=jnp.float32)
    utab4 = _tc_pack(user_table.T, eye32)
    itab4 = _tc_pack(item_table.T, eye32)
    uflat, vflat = _sc_gather2(user_ids.astype(jnp.int32),
                               item_ids.astype(jnp.int32), utab4, itab4)
    u4 = uflat.reshape(BP, PK * D)
    v4 = vflat.reshape(BP, PK * D)
    eye = jnp.eye(PK, dtype=jnp.float32)
    o4 = _tc_mlp(u4, v4, jnp.kron(eye, W1[:D]), jnp.kron(eye, W1[D:]),
                 jnp.tile(b1, PK).reshape(1, PK * H1),
                 jnp.kron(eye, W2), jnp.tile(b2, PK).reshape(1, PK * H2),
                 jnp.kron(eye, W3), jnp.tile(b3, PK).reshape(1, PK))
    return o4.reshape(BATCH, 1)


# speedup vs baseline: 1.6046x; 1.0153x over previous
import functools

import jax
import jax.numpy as jnp
from jax import lax
from jax.experimental import pallas as pl
from jax.experimental.pallas import tpu as pltpu
from jax.experimental.pallas import tpu_sc as plsc

BATCH = 16384
D = 32
H1 = 64
H2 = 32
PK = 4
BP = BATCH // PK
TROWS = 1000000 // PK

NC = 2
NS = 16
NW = NC * NS
B_PER_W = BATCH // NW
R_PER_W = B_PER_W // PK
CHUNK = 128

_mesh = plsc.VectorSubcoreMesh(
    core_axis_name="c", subcore_axis_name="s", num_cores=NC, num_subcores=NS
)


@functools.partial(
    pl.kernel,
    out_type=(
        jax.ShapeDtypeStruct((BP * PK * D,), jnp.float32),
        jax.ShapeDtypeStruct((BP * PK * D,), jnp.float32),
    ),
    mesh=_mesh,
    compiler_params=pltpu.CompilerParams(use_tc_tiling_on_sc=True,
                                         needs_layout_passes=False),
    scratch_types=[
        pltpu.VMEM((B_PER_W,), jnp.int32),
        pltpu.VMEM((B_PER_W,), jnp.int32),
        pltpu.VMEM((B_PER_W,), jnp.int32),
        pltpu.VMEM((B_PER_W, PK * D), jnp.float32),
        pltpu.VMEM((B_PER_W * D,), jnp.float32),
        pltpu.SemaphoreType.DMA,
    ],
)
def _sc_gather2(uids_hbm, iids_hbm, utab_hbm, itab_hbm, uout_hbm, iout_hbm,
                ids_v, gidx_v, lane_v, gath_v, outst_v, sem):
    wid = lax.axis_index("s") * NC + lax.axis_index("c")
    base = wid * B_PER_W
    obase = wid * B_PER_W * D

    iota16 = lax.iota(jnp.int32, 16)

    def one_table(ids_hbm, tab_hbm, out_hbm):
        pltpu.sync_copy(ids_hbm.at[pl.ds(base, B_PER_W)], ids_v)

        def idx_body(g, _):
            goff = g * 16
            ids = ids_v[pl.ds(goff, 16)]
            gidx_v[pl.ds(goff, 16)] = lax.bitwise_or(
                lax.shift_left(lax.shift_right_logical(ids, 14), 12),
                lax.bitwise_and(ids, 4095))
            lane_v[pl.ds(goff, 16)] = lax.bitwise_and(
                lax.shift_right_logical(ids, 12), 3)
            return 0

        lax.fori_loop(0, B_PER_W // 16, idx_body, 0)

        copies = [
            pltpu.async_copy(tab_hbm.at[gidx_v.at[pl.ds(k * CHUNK, CHUNK)]],
                             gath_v.at[pl.ds(k * CHUNK, CHUNK)], sem)
            for k in range(B_PER_W // CHUNK)
        ]
        for cp in copies:
            cp.wait()

        def ext_body(g, _):
            flat = g * 16 + iota16
            m = lax.shift_right_logical(flat, 5)
            e = lax.bitwise_and(flat, 31)
            ln = plsc.load_gather(lane_v, [m])
            val = plsc.load_gather(gath_v, [m, ln * D + e])
            outst_v[pl.ds(g * 16, 16)] = val
            return 0

        lax.fori_loop(0, (B_PER_W * D) // 16, ext_body, 0)
        pltpu.sync_copy(outst_v, out_hbm.at[pl.ds(obase, B_PER_W * D)])

    one_table(uids_hbm, utab_hbm, uout_hbm)
    one_table(iids_hbm, itab_hbm, iout_hbm)


def _mlp_body(u_ref, v_ref, w1a_ref, w1b_ref, b1_ref, w2_ref, b2_ref,
              w3_ref, b3_ref, o_ref):
    h = (jnp.dot(u_ref[...], w1a_ref[...], preferred_element_type=jnp.float32)
         + jnp.dot(v_ref[...], w1b_ref[...], preferred_element_type=jnp.float32)
         + b1_ref[...])
    h = jnp.maximum(h, 0.0)
    h = jnp.maximum(
        jnp.dot(h, w2_ref[...], preferred_element_type=jnp.float32) + b2_ref[...],
        0.0)
    o_ref[...] = (jnp.dot(h, w3_ref[...], preferred_element_type=jnp.float32)
                  + b3_ref[...])


_TW = 16384
_TGRID = 62
TROWS_PAD = _TGRID * (_TW // PK)


def _pack_body(x_ref, eye_ref, o_ref):
    x = x_ref[...]
    eye = eye_ref[...]
    q = _TW // PK
    o_ref[...] = jnp.concatenate(
        [lax.dot_general(x[:, k * q:(k + 1) * q], eye, (((0,), (0,)), ((), ())),
                         preferred_element_type=jnp.float32)
         for k in range(PK)], axis=1)


def _tc_pack(tab_t, eye32):
    return pl.pallas_call(
        _pack_body,
        grid=(_TGRID,),
        in_specs=[pl.BlockSpec((D, _TW), lambda i: (0, i)),
                  pl.BlockSpec((D, D), lambda i: (0, 0))],
        out_specs=pl.BlockSpec((_TW // PK, PK * D), lambda i: (i, 0)),
        out_shape=jax.ShapeDtypeStruct((TROWS_PAD, PK * D), jnp.float32),
    )(tab_t, eye32)


_BLK = 1024


def _tc_mlp(u, v, w1a, w1b, b1, w2, b2, w3, b3):
    grid = (BP // _BLK,)
    full = lambda i: (0, 0)
    return pl.pallas_call(
        _mlp_body,
        grid=grid,
        in_specs=[
            pl.BlockSpec((_BLK, PK * D), lambda i: (i, 0)),
            pl.BlockSpec((_BLK, PK * D), lambda i: (i, 0)),
            pl.BlockSpec((PK * D, PK * H1), full),
            pl.BlockSpec((PK * D, PK * H1), full),
            pl.BlockSpec((1, PK * H1), full),
            pl.BlockSpec((PK * H1, PK * H2), full),
            pl.BlockSpec((1, PK * H2), full),
            pl.BlockSpec((PK * H2, PK), full),
            pl.BlockSpec((1, PK), full),
        ],
        out_specs=pl.BlockSpec((_BLK, PK), lambda i: (i, 0)),
        out_shape=jax.ShapeDtypeStruct((BP, PK), jnp.float32),
    )(u, v, w1a, w1b, b1, w2, b2, w3, b3)


def kernel(user_ids, item_ids, user_table, item_table, W1, b1, W2, b2, W3, b3):
    eye32 = jnp.eye(D, dtype=jnp.float32)
    utab4 = _tc_pack(user_table.T, eye32)
    itab4 = _tc_pack(item_table.T, eye32)
    uflat, vflat = _sc_gather2(user_ids.astype(jnp.int32),
                               item_ids.astype(jnp.int32), utab4, itab4)
    u4 = uflat.reshape(BP, PK * D)
    v4 = vflat.reshape(BP, PK * D)
    eye = jnp.eye(PK, dtype=jnp.float32)
    o4 = _tc_mlp(u4, v4, jnp.kron(eye, W1[:D]), jnp.kron(eye, W1[D:]),
                 jnp.tile(b1, PK).reshape(1, PK * H1),
                 jnp.kron(eye, W2), jnp.tile(b2, PK).reshape(1, PK * H2),
                 jnp.kron(eye, W3), jnp.tile(b3, PK).reshape(1, PK))
    return o4.reshape(BATCH, 1)
